# baseline (speedup 1.0000x reference)
import functools

import jax
import jax.numpy as jnp
from jax import lax
from jax.experimental import pallas as pl
from jax.experimental.pallas import tpu as pltpu
from jax.experimental.pallas import tpu_sc as plsc

B, C, H, W = 8, 384, 224, 224
P = 200
HW = H * W
CHW = C * HW
NPTS = 2 * B * P
NW = 32
PPW = NPTS // NW
ROWS_PER_PT = C // 128
NROWS = PPW * ROWS_PER_PT
LANES = 16


def _body(img_hbm, xs_hbm, ys_hbm, out_hbm, xs_v, ys_v, base_v, idx_v,
          gat_v, sem):
    wid = lax.axis_index("s") * 2 + lax.axis_index("c")
    b = (wid % 16) // 2

    pltpu.sync_copy(xs_hbm.at[wid], xs_v)
    pltpu.sync_copy(ys_hbm.at[wid], ys_v)

    iota = lax.iota(jnp.int32, LANES)

    for k in range(128 // LANES):
        xv = xs_v[pl.ds(k * LANES, LANES)]
        yv = ys_v[pl.ds(k * LANES, LANES)]
        base_v[pl.ds(k * LANES, LANES)] = b * CHW + yv * W + xv

    def fill(p, carry):
        bp = base_v[pl.ds(p, LANES)][0]
        for cb in range(C // LANES):
            row = ROWS_PER_PT * p + cb // 8
            col = LANES * (cb % 8)
            idx_v[row, pl.ds(col, LANES)] = bp + (cb * LANES + iota) * HW
        return carry

    lax.fori_loop(0, PPW, fill, 0)

    def fire(j, carry):
        pltpu.async_copy(img_hbm.at[idx_v.at[j]],
                         gat_v.at[pl.ds(j * 128, 128)], sem)
        return carry

    lax.fori_loop(0, NROWS, fire, 0)

    def drain(j, carry):
        pltpu.make_async_copy(img_hbm.at[pl.ds(0, 128)],
                              gat_v.at[pl.ds(0, 128)], sem).wait()
        return carry

    lax.fori_loop(0, NROWS, drain, 0)

    pltpu.sync_copy(gat_v, out_hbm.at[pl.ds(wid * (NROWS * 128),
                                            NROWS * 128)])


@jax.jit
def _crop_gather(img_flat, xs_pad, ys_pad):
    kern = functools.partial(
        pl.kernel,
        out_type=jax.ShapeDtypeStruct((NW * NROWS * 128,), jnp.float32),
        mesh=plsc.VectorSubcoreMesh(core_axis_name="c",
                                    subcore_axis_name="s"),
        scratch_types=[
            pltpu.VMEM((128,), jnp.int32),
            pltpu.VMEM((128,), jnp.int32),
            pltpu.VMEM((128,), jnp.int32),
            pltpu.VMEM((NROWS, 128), jnp.int32),
            pltpu.VMEM((NROWS * 128,), jnp.float32),
            pltpu.SemaphoreType.DMA,
        ],
    )(_body)
    return kern(img_flat, xs_pad, ys_pad)


def kernel(imgs, batch_cpoints, batch_npoints):
    img_flat = imgs.reshape(-1)
    pts = jnp.concatenate(
        [batch_cpoints.reshape(-1, 2), batch_npoints.reshape(-1, 2)], axis=0)
    xs = pts[:, 0].reshape(NW, PPW)
    ys = pts[:, 1].reshape(NW, PPW)
    pad = ((0, 0), (0, 128 - PPW))
    xs_pad = jnp.pad(xs, pad)
    ys_pad = jnp.pad(ys, pad)
    out = _crop_gather(img_flat, xs_pad, ys_pad)
    batch_crop_imgs = out.reshape(NPTS, C)
    return (batch_crop_imgs, NPTS // 2, NPTS)
